# Initial kernel scaffold; baseline (speedup 1.0000x reference)
import jax
import jax.numpy as jnp
from jax.experimental import pallas as pl


def kernel(inputs, indices, w, b):
    raise NotImplementedError("write your pallas kernel here")


# speedup vs baseline: 10.8717x; 10.8717x over previous
import jax
import jax.numpy as jnp
from jax.experimental import pallas as pl
from jax.experimental.pallas import tpu as pltpu


def _moe_kernel(idx_ref, x_ref, w_ref, b_ref, out_ref):
    e = pl.program_id(0)
    mask = (idx_ref[...] == e).astype(jnp.float32)
    xm = x_ref[...] * mask
    acc = jnp.dot(xm, w_ref[0], preferred_element_type=jnp.float32)
    acc = acc + mask * b_ref[0]

    @pl.when(e == 0)
    def _init():
        out_ref[...] = acc

    @pl.when(e != 0)
    def _accum():
        out_ref[...] += acc


def kernel(inputs, indices, w, b):
    N, D = inputs.shape
    E, _, O = w.shape
    idx2d = indices.astype(jnp.int32).reshape(N, 1)
    b3d = b.reshape(E, 1, O)

    return pl.pallas_call(
        _moe_kernel,
        grid=(E,),
        in_specs=[
            pl.BlockSpec((N, 1), lambda e: (0, 0)),
            pl.BlockSpec((N, D), lambda e: (0, 0)),
            pl.BlockSpec((1, D, O), lambda e: (e, 0, 0)),
            pl.BlockSpec((1, 1, O), lambda e: (e, 0, 0)),
        ],
        out_specs=pl.BlockSpec((N, O), lambda e: (0, 0)),
        out_shape=jax.ShapeDtypeStruct((N, O), jnp.float32),
    )(idx2d, inputs, w, b3d)


# speedup vs baseline: 11.7001x; 1.0762x over previous
import jax
import jax.numpy as jnp
from jax.experimental import pallas as pl
from jax.experimental.pallas import tpu as pltpu

_NBUF = 4
_C = 2


def _moe_kernel(idx_ref, x_ref, w_hbm, b_ref, out_ref, w_buf, sem):
    E, D, O = w_hbm.shape
    DC = D // _C
    TOT = E * _C

    def make_copy(t, slot):
        e = t // _C
        c = jax.lax.rem(t, _C)
        return pltpu.make_async_copy(
            w_hbm.at[e, pl.ds(c * DC, DC), :],
            w_buf.at[slot],
            sem.at[slot],
        )

    for s in range(_NBUF):
        make_copy(s, s).start()

    def body(r, _):
        for s in range(_NBUF):
            t = r * _NBUF + s
            e = t // _C
            c = jax.lax.rem(t, _C)
            make_copy(t, s).wait()
            mask = (idx_ref[...] == e).astype(jnp.float32)
            xm = x_ref[:, pl.ds(c * DC, DC)] * mask
            part = jnp.dot(xm, w_buf[s], preferred_element_type=jnp.float32)
            part = jnp.where(c == _C - 1, part + mask * b_ref[e], part)

            @pl.when(t == 0)
            def _init():
                out_ref[...] = part

            @pl.when(t != 0)
            def _accum():
                out_ref[...] += part

            nxt = t + _NBUF

            @pl.when(nxt < TOT)
            def _prefetch():
                make_copy(nxt, s).start()

        return 0

    jax.lax.fori_loop(0, TOT // _NBUF, body, 0)


def kernel(inputs, indices, w, b):
    N, D = inputs.shape
    E, _, O = w.shape
    idx2d = indices.astype(jnp.int32).reshape(N, 1)
    b3d = b.reshape(E, 1, O)

    return pl.pallas_call(
        _moe_kernel,
        in_specs=[
            pl.BlockSpec(memory_space=pltpu.VMEM),
            pl.BlockSpec(memory_space=pltpu.VMEM),
            pl.BlockSpec(memory_space=pl.ANY),
            pl.BlockSpec(memory_space=pltpu.VMEM),
        ],
        out_specs=pl.BlockSpec(memory_space=pltpu.VMEM),
        out_shape=jax.ShapeDtypeStruct((N, O), jnp.float32),
        scratch_shapes=[
            pltpu.VMEM((_NBUF, D // _C, O), jnp.float32),
            pltpu.SemaphoreType.DMA((_NBUF,)),
        ],
    )(idx2d, inputs, w, b3d)
